# baseline (speedup 1.0000x reference)
import jax
import jax.numpy as jnp
from jax import lax
from jax.experimental import pallas as pl
from jax.experimental.pallas import tpu as pltpu
from jax.experimental.pallas import tpu_sc as plsc

N = 10000
E = 320000
NPAD = 10240
NACC = 10016
NC, NS, L = 2, 16, 16
NW = NC * NS
C = 112
KCH = 94
KAL = KCH + 2
P = NW * C * KCH


def _tc_prep_body(x_ref, w_ref, gc_ref, gd_ref, th_ref, td_ref):
    h = jnp.dot(x_ref[...], w_ref[...], preferred_element_type=jnp.float32)
    th_ref[...] = jnp.dot(h, gc_ref[...], preferred_element_type=jnp.float32)
    td_ref[...] = jnp.dot(h, gd_ref[...], preferred_element_type=jnp.float32)


def _tc_prep(xp, W, Gc, Gd):
    F = xp.shape[1]
    DH = W.shape[1]
    DW = Gc.shape[1]
    BN = 2048
    grid = (NPAD // BN,)
    return pl.pallas_call(
        _tc_prep_body,
        grid=grid,
        in_specs=[
            pl.BlockSpec((BN, F), lambda i: (i, 0)),
            pl.BlockSpec((F, DH), lambda i: (0, 0)),
            pl.BlockSpec((DH, DW), lambda i: (0, 0)),
            pl.BlockSpec((DH, 16), lambda i: (0, 0)),
        ],
        out_specs=[
            pl.BlockSpec((BN, DW), lambda i: (i, 0)),
            pl.BlockSpec((BN, 16), lambda i: (i, 0)),
        ],
        out_shape=[
            jax.ShapeDtypeStruct((NPAD, DW), jnp.float32),
            jax.ShapeDtypeStruct((NPAD, 16), jnp.float32),
        ],
    )(xp, W, Gc, Gd)


def _tc_mid_body(ua_ref, ub_ref, r_ref, b_ref, w_ref, gc_ref, gd_ref,
                 th_ref, td_ref):
    d1 = r_ref.shape[1]
    u = ua_ref[...] + ub_ref[...]
    den = jnp.dot(u[:, d1:d1 + 16], r_ref[...],
                  preferred_element_type=jnp.float32)
    hpre = u[:, :d1] / (den + 1e-16) + b_ref[...]
    h = jnp.where(hpre > 0, hpre, jnp.exp(jnp.minimum(hpre, 0.0)) - 1.0)
    h2 = jnp.dot(h, w_ref[...], preferred_element_type=jnp.float32)
    th_ref[...] = jnp.dot(h2, gc_ref[...], preferred_element_type=jnp.float32)
    td_ref[...] = jnp.dot(h2, gd_ref[...], preferred_element_type=jnp.float32)


def _tc_mid(Ua, Ub, R, b1r, W2, Gc, Gd):
    DW1 = Ua.shape[1]
    d1 = R.shape[1]
    F2 = W2.shape[1]
    DW2 = Gc.shape[1]
    BN = 2048
    grid = (NPAD // BN,)
    return pl.pallas_call(
        _tc_mid_body,
        grid=grid,
        in_specs=[
            pl.BlockSpec((BN, DW1), lambda i: (i, 0)),
            pl.BlockSpec((BN, DW1), lambda i: (i, 0)),
            pl.BlockSpec((16, d1), lambda i: (0, 0)),
            pl.BlockSpec((1, d1), lambda i: (0, 0)),
            pl.BlockSpec((d1, F2), lambda i: (0, 0)),
            pl.BlockSpec((F2, DW2), lambda i: (0, 0)),
            pl.BlockSpec((F2, 16), lambda i: (0, 0)),
        ],
        out_specs=[
            pl.BlockSpec((BN, DW2), lambda i: (i, 0)),
            pl.BlockSpec((BN, 16), lambda i: (i, 0)),
        ],
        out_shape=[
            jax.ShapeDtypeStruct((NPAD, DW2), jnp.float32),
            jax.ShapeDtypeStruct((NPAD, 16), jnp.float32),
        ],
    )(Ua, Ub, R, b1r, W2, Gc, Gd)


def _tc_final_body(ua_ref, ub_ref, r_ref, b_ref, o_ref):
    d2 = r_ref.shape[1]
    u = ua_ref[...] + ub_ref[...]
    den = jnp.dot(u[:, d2:d2 + 16], r_ref[...],
                  preferred_element_type=jnp.float32)
    o_ref[...] = u[:, :d2] / (den + 1e-16) + b_ref[...]


def _tc_final(Ua, Ub, R2, b2r):
    DW = Ua.shape[1]
    d2 = R2.shape[1]
    BN = 2000
    grid = (N // BN,)
    return pl.pallas_call(
        _tc_final_body,
        grid=grid,
        in_specs=[
            pl.BlockSpec((BN, DW), lambda i: (i, 0)),
            pl.BlockSpec((BN, DW), lambda i: (i, 0)),
            pl.BlockSpec((16, d2), lambda i: (0, 0)),
            pl.BlockSpec((1, d2), lambda i: (0, 0)),
        ],
        out_specs=pl.BlockSpec((BN, d2), lambda i: (i, 0)),
        out_shape=jax.ShapeDtypeStruct((N, d2), jnp.float32),
    )(Ua, Ub, R2, b2r)


def _make_edge_pass(DM, out_ch):
    DW = DM + 16
    NQ = DM // L
    ZR = NACC // NS
    sh = out_ch.bit_length() - 1

    def body(th, tad, edges, zdm,
             ua, ub,
             idx0, idx1, rows0, rows1, drows0, drows1, dsti0, dsti1,
             u_sh,
             si0, si1, gr0, gr1, gd0, gd1, su0, su1):
        cid = lax.axis_index("c")
        sid = lax.axis_index("s")
        w = cid * NS + sid

        pltpu.sync_copy(zdm, u_sh.at[pl.ds(sid * ZR, ZR)])
        plsc.subcore_barrier()

        lane = lax.iota(jnp.int32, L)
        idx_vecs = [(lane + q * L) >> sh for q in range(NQ)]

        bufs = ((idx0, rows0, drows0, dsti0, si0, gr0, gd0, su0),
                (idx1, rows1, drows1, dsti1, si1, gr1, gd1, su1))

        def idx_copy(k, b):
            idx_v, _, _, _, si, *_ = bufs[b]
            return pltpu.make_async_copy(edges.at[w * KAL + k], idx_v, si)

        def gather_copies(b):
            idx_v, rows_v, drows_v, _, _, gr, gd, _ = bufs[b]
            c1 = pltpu.make_async_copy(th.at[idx_v.at[0]], rows_v, gr)
            c2 = pltpu.make_async_copy(tad.at[idx_v.at[1]], drows_v, gd)
            return c1, c2

        def scatter_copies(b):
            _, rows_v, _, dsti, _, _, _, su = bufs[b]
            return (pltpu.make_async_copy(rows_v, u_sh.at[dsti], su),)

        def compute(b):
            idx_v, rows_v, drows_v, dsti, *_ = bufs[b]
            for i in range(C // L):
                dsti[pl.ds(i * L, L)] = idx_v[1, pl.ds(i * L, L)]

            def edge(e, carry):
                asrc = rows_v[e, pl.ds(DM, L)]
                adst = drows_v[e, pl.ds(0, L)]
                s = asrc + adst
                s = jnp.where(s >= 0, s, 0.2 * s)
                t = jnp.exp(s)
                rows_v[e, pl.ds(DM, L)] = t
                for q in range(NQ):
                    tb = jnp.take_along_axis(t, idx_vecs[q], axis=0)
                    rows_v[e, pl.ds(q * L, L)] = (
                        rows_v[e, pl.ds(q * L, L)] * tb)
                return carry

            lax.fori_loop(0, C, edge, 0, unroll=2)

        def phase(k, b, first):
            nb = 1 - b
            idx_copy(k + 1, nb).wait()
            if not first:
                for c in scatter_copies(nb):
                    c.wait()
            for c in gather_copies(nb):
                c.start()
            for c in gather_copies(b):
                c.wait()
            compute(b)
            idx_copy(k + 2, b).start()
            for c in scatter_copies(b):
                c.start(add=True)

        pltpu.sync_copy(edges.at[w * KAL], idx0)
        for c in gather_copies(0):
            c.start()
        idx_copy(1, 1).start()
        phase(0, 0, True)
        phase(1, 1, False)

        def pair(k2, carry):
            phase(2 * k2, 0, False)
            phase(2 * k2 + 1, 1, False)
            return carry

        lax.fori_loop(1, KCH // 2, pair, 0)

        idx_copy(KCH + 1, 1).wait()
        for c in gather_copies(0):
            c.wait()
        for c in scatter_copies(1):
            c.wait()

        plsc.subcore_barrier()

        rs = pl.ds(sid * ZR, ZR)

        @pl.when(cid == 0)
        def _():
            pltpu.sync_copy(u_sh.at[rs], ua.at[rs])

        @pl.when(cid == 1)
        def _():
            pltpu.sync_copy(u_sh.at[rs], ub.at[rs])

    mesh = plsc.VectorSubcoreMesh(core_axis_name="c", subcore_axis_name="s",
                                  num_cores=NC, num_subcores=NS)
    return pl.kernel(
        body,
        out_type=[
            jax.ShapeDtypeStruct((NPAD, DW), jnp.float32),
            jax.ShapeDtypeStruct((NPAD, DW), jnp.float32),
        ],
        mesh=mesh,
        compiler_params=pltpu.CompilerParams(use_tc_tiling_on_sc=False),
        scratch_types=[
            pltpu.VMEM((2, C), jnp.int32),
            pltpu.VMEM((2, C), jnp.int32),
            pltpu.VMEM((C, DW), jnp.float32),
            pltpu.VMEM((C, DW), jnp.float32),
            pltpu.VMEM((C, 16), jnp.float32),
            pltpu.VMEM((C, 16), jnp.float32),
            pltpu.VMEM((C,), jnp.int32),
            pltpu.VMEM((C,), jnp.int32),
            pltpu.MemorySpace.VMEM_SHARED((NACC, DW), jnp.float32),
        ] + [pltpu.SemaphoreType.DMA] * 8,
    )


_edge_pass_1 = _make_edge_pass(64, 8)
_edge_pass_2 = _make_edge_pass(128, 128)


def kernel(x, edge_index, W1, att_src1, att_dst1, b1, W2, att_src2,
           att_dst2, b2):
    f32 = jnp.float32
    heads1, ch1 = att_src1.shape
    d1 = heads1 * ch1

    xp = jnp.zeros((NPAD, x.shape[1]), f32).at[:N].set(x)
    loops = jnp.arange(N, dtype=jnp.int32)
    pad_idx = jnp.full((P - E - N,), N, dtype=jnp.int32)
    src = jnp.concatenate([edge_index[0].astype(jnp.int32), loops, pad_idx])
    dst = jnp.concatenate([edge_index[1].astype(jnp.int32), loops, pad_idx])
    sd = jnp.stack([src.reshape(NW, KCH, C), dst.reshape(NW, KCH, C)],
                   axis=2)
    tail = jnp.full((NW, KAL - KCH, 2, C), N, dtype=jnp.int32)
    edges = jnp.concatenate([sd, tail], axis=1).reshape(NW * KAL, 2, C)

    eye_h = jnp.eye(heads1, dtype=f32)
    m_src1 = (att_src1[:, :, None] * eye_h[:, None, :]).reshape(d1, heads1)
    m_dst1 = (att_dst1[:, :, None] * eye_h[:, None, :]).reshape(d1, heads1)
    g1c = jnp.concatenate(
        [jnp.eye(d1, dtype=f32), m_src1, jnp.zeros((d1, 16 - heads1), f32)],
        axis=1)
    g1d = jnp.concatenate([m_dst1, jnp.zeros((d1, 16 - heads1), f32)], axis=1)

    d2 = W2.shape[1]
    g2c = jnp.concatenate(
        [jnp.eye(d2, dtype=f32), att_src2.T, jnp.zeros((d2, 15), f32)],
        axis=1)
    g2d = jnp.concatenate([att_dst2.T, jnp.zeros((d2, 15), f32)], axis=1)

    r1 = jnp.concatenate(
        [jnp.repeat(eye_h, ch1, axis=1), jnp.zeros((16 - heads1, d1), f32)],
        axis=0)
    r2 = jnp.zeros((16, d2), f32).at[0].set(1.0)

    z80 = jnp.zeros((NACC // NS, d1 + 16), f32)
    z144 = jnp.zeros((NACC // NS, d2 + 16), f32)

    th1, tad1 = _tc_prep(xp, W1, g1c, g1d)
    ua1, ub1 = _edge_pass_1(th1, tad1, edges, z80)

    th2, tad2 = _tc_mid(ua1, ub1, r1, b1.reshape(1, d1), W2, g2c, g2d)
    ua2, ub2 = _edge_pass_2(th2, tad2, edges, z144)

    return _tc_final(ua2[:N], ub2[:N], r2, b2.reshape(1, d2))


# speedup vs baseline: 1.2888x; 1.2888x over previous
import jax
import jax.numpy as jnp
from jax import lax
from jax.experimental import pallas as pl
from jax.experimental.pallas import tpu as pltpu
from jax.experimental.pallas import tpu_sc as plsc

N = 10000
E = 320000
NPAD = 10240
NACC = 10016
NC, NS, L = 2, 16, 16
NW = NC * NS
C = 96
KCH = 108
KAL = KCH + 2
P = NW * C * KCH


def _tc_prep_body(x_ref, w_ref, gc_ref, gd_ref, th_ref, td_ref):
    h = jnp.dot(x_ref[...], w_ref[...], preferred_element_type=jnp.float32)
    th_ref[...] = jnp.dot(h, gc_ref[...], preferred_element_type=jnp.float32)
    td_ref[...] = jnp.dot(h, gd_ref[...], preferred_element_type=jnp.float32)


def _tc_prep(xp, W, Gc, Gd):
    F = xp.shape[1]
    DH = W.shape[1]
    DW = Gc.shape[1]
    BN = 2048
    grid = (NPAD // BN,)
    return pl.pallas_call(
        _tc_prep_body,
        grid=grid,
        in_specs=[
            pl.BlockSpec((BN, F), lambda i: (i, 0)),
            pl.BlockSpec((F, DH), lambda i: (0, 0)),
            pl.BlockSpec((DH, DW), lambda i: (0, 0)),
            pl.BlockSpec((DH, 16), lambda i: (0, 0)),
        ],
        out_specs=[
            pl.BlockSpec((BN, DW), lambda i: (i, 0)),
            pl.BlockSpec((BN, 16), lambda i: (i, 0)),
        ],
        out_shape=[
            jax.ShapeDtypeStruct((NPAD, DW), jnp.float32),
            jax.ShapeDtypeStruct((NPAD, 16), jnp.float32),
        ],
    )(xp, W, Gc, Gd)


def _tc_mid_body(ua_ref, ub_ref, r_ref, b_ref, w_ref, gc_ref, gd_ref,
                 th_ref, td_ref):
    d1 = r_ref.shape[1]
    u = ua_ref[...] + ub_ref[...]
    den = jnp.dot(u[:, d1:d1 + 16], r_ref[...],
                  preferred_element_type=jnp.float32)
    hpre = u[:, :d1] / (den + 1e-16) + b_ref[...]
    h = jnp.where(hpre > 0, hpre, jnp.exp(jnp.minimum(hpre, 0.0)) - 1.0)
    h2 = jnp.dot(h, w_ref[...], preferred_element_type=jnp.float32)
    th_ref[...] = jnp.dot(h2, gc_ref[...], preferred_element_type=jnp.float32)
    td_ref[...] = jnp.dot(h2, gd_ref[...], preferred_element_type=jnp.float32)


def _tc_mid(Ua, Ub, R, b1r, W2, Gc, Gd):
    DW1 = Ua.shape[1]
    d1 = R.shape[1]
    F2 = W2.shape[1]
    DW2 = Gc.shape[1]
    BN = 2048
    grid = (NPAD // BN,)
    return pl.pallas_call(
        _tc_mid_body,
        grid=grid,
        in_specs=[
            pl.BlockSpec((BN, DW1), lambda i: (i, 0)),
            pl.BlockSpec((BN, DW1), lambda i: (i, 0)),
            pl.BlockSpec((16, d1), lambda i: (0, 0)),
            pl.BlockSpec((1, d1), lambda i: (0, 0)),
            pl.BlockSpec((d1, F2), lambda i: (0, 0)),
            pl.BlockSpec((F2, DW2), lambda i: (0, 0)),
            pl.BlockSpec((F2, 16), lambda i: (0, 0)),
        ],
        out_specs=[
            pl.BlockSpec((BN, DW2), lambda i: (i, 0)),
            pl.BlockSpec((BN, 16), lambda i: (i, 0)),
        ],
        out_shape=[
            jax.ShapeDtypeStruct((NPAD, DW2), jnp.float32),
            jax.ShapeDtypeStruct((NPAD, 16), jnp.float32),
        ],
    )(Ua, Ub, R, b1r, W2, Gc, Gd)


def _tc_final_body(ua_ref, ub_ref, r_ref, b_ref, o_ref):
    d2 = r_ref.shape[1]
    u = ua_ref[...] + ub_ref[...]
    den = jnp.dot(u[:, d2:d2 + 16], r_ref[...],
                  preferred_element_type=jnp.float32)
    o_ref[...] = u[:, :d2] / (den + 1e-16) + b_ref[...]


def _tc_final(Ua, Ub, R2, b2r):
    DW = Ua.shape[1]
    d2 = R2.shape[1]
    BN = 2000
    grid = (N // BN,)
    return pl.pallas_call(
        _tc_final_body,
        grid=grid,
        in_specs=[
            pl.BlockSpec((BN, DW), lambda i: (i, 0)),
            pl.BlockSpec((BN, DW), lambda i: (i, 0)),
            pl.BlockSpec((16, d2), lambda i: (0, 0)),
            pl.BlockSpec((1, d2), lambda i: (0, 0)),
        ],
        out_specs=pl.BlockSpec((BN, d2), lambda i: (i, 0)),
        out_shape=jax.ShapeDtypeStruct((N, d2), jnp.float32),
    )(Ua, Ub, R2, b2r)


def _make_edge_pass(DM, out_ch):
    DW = DM + 16
    NQ = DM // L
    ZR = NACC // NS
    sh = out_ch.bit_length() - 1

    def body(th, tad, edges, zdm,
             ua, ub,
             idx0, idx1, idx2, rows0, rows1, drows0, drows1,
             u_sh,
             si0, si1, si2, gr0, gr1, gd0, gd1, su0, su1):
        cid = lax.axis_index("c")
        sid = lax.axis_index("s")
        w = cid * NS + sid

        pltpu.sync_copy(zdm, u_sh.at[pl.ds(sid * ZR, ZR)])
        plsc.subcore_barrier()

        lane = lax.iota(jnp.int32, L)
        pat_key = [tuple((l + q * L) >> sh for l in range(L))
                   for q in range(NQ)]
        pat_rep = {}
        for q in range(NQ):
            pat_rep.setdefault(pat_key[q], q)
        idx_vecs = {k: (lane + q0 * L) >> sh for k, q0 in pat_rep.items()}

        idxb = (idx0, idx1, idx2)
        sib = (si0, si1, si2)
        rowsb = (rows0, rows1)
        drowsb = (drows0, drows1)
        grb = (gr0, gr1)
        gdb = (gd0, gd1)
        sub = (su0, su1)

        def idx_copy(k, j):
            return pltpu.make_async_copy(edges.at[w * KAL + k], idxb[j],
                                         sib[j])

        def gather_copies(rb, j):
            c1 = pltpu.make_async_copy(th.at[idxb[j].at[0]], rowsb[rb],
                                       grb[rb])
            c2 = pltpu.make_async_copy(tad.at[idxb[j].at[1]], drowsb[rb],
                                       gdb[rb])
            return c1, c2

        def scatter_copy(rb, j):
            return pltpu.make_async_copy(rowsb[rb],
                                         u_sh.at[idxb[j].at[1]], sub[rb])

        def compute(rb):
            rows_v = rowsb[rb]
            drows_v = drowsb[rb]

            def edge(e, carry):
                asrc = rows_v[e, pl.ds(DM, L)]
                adst = drows_v[e, pl.ds(0, L)]
                s = asrc + adst
                s = jnp.maximum(s, 0.2 * s)
                t = jnp.exp(s)
                rows_v[e, pl.ds(DM, L)] = t
                tbs = {k: jnp.take_along_axis(t, v, axis=0)
                       for k, v in idx_vecs.items()}
                for q in range(NQ):
                    rows_v[e, pl.ds(q * L, L)] = (
                        rows_v[e, pl.ds(q * L, L)] * tbs[pat_key[q]])
                return carry

            lax.fori_loop(0, C, edge, 0, unroll=4)

        def phase(kd, p, first):
            rb, ib = p % 2, p % 3
            nrb, ib1, ib2 = 1 - rb, (ib + 1) % 3, (ib + 2) % 3
            idx_copy(kd + 1, ib1).wait()
            if not first:
                scatter_copy(nrb, ib2).wait()
            for c in gather_copies(nrb, ib1):
                c.start()
            for c in gather_copies(rb, ib):
                c.wait()
            compute(rb)
            idx_copy(kd + 2, ib2).start()
            scatter_copy(rb, ib).start(add=True)

        pltpu.sync_copy(edges.at[w * KAL], idx0)
        for c in gather_copies(0, 0):
            c.start()
        idx_copy(1, 1).start()
        phase(0, 0, True)
        for p in range(1, 6):
            phase(p, p, False)

        def six(j, carry):
            for p in range(6):
                phase(6 * j + p, p, False)
            return carry

        lax.fori_loop(1, KCH // 6, six, 0)

        last = KCH - 1
        rbl, ibl = last % 2, last % 3
        idx_copy(KCH + 1, (ibl + 2) % 3).wait()
        for c in gather_copies(1 - rbl, (ibl + 1) % 3):
            c.wait()
        scatter_copy(rbl, ibl).wait()

        plsc.subcore_barrier()

        rs = pl.ds(sid * ZR, ZR)

        @pl.when(cid == 0)
        def _():
            pltpu.sync_copy(u_sh.at[rs], ua.at[rs])

        @pl.when(cid == 1)
        def _():
            pltpu.sync_copy(u_sh.at[rs], ub.at[rs])

    mesh = plsc.VectorSubcoreMesh(core_axis_name="c", subcore_axis_name="s",
                                  num_cores=NC, num_subcores=NS)
    return pl.kernel(
        body,
        out_type=[
            jax.ShapeDtypeStruct((NPAD, DW), jnp.float32),
            jax.ShapeDtypeStruct((NPAD, DW), jnp.float32),
        ],
        mesh=mesh,
        compiler_params=pltpu.CompilerParams(use_tc_tiling_on_sc=False),
        scratch_types=[
            pltpu.VMEM((2, C), jnp.int32),
            pltpu.VMEM((2, C), jnp.int32),
            pltpu.VMEM((2, C), jnp.int32),
            pltpu.VMEM((C, DW), jnp.float32),
            pltpu.VMEM((C, DW), jnp.float32),
            pltpu.VMEM((C, 16), jnp.float32),
            pltpu.VMEM((C, 16), jnp.float32),
            pltpu.MemorySpace.VMEM_SHARED((NACC, DW), jnp.float32),
        ] + [pltpu.SemaphoreType.DMA] * 9,
    )


_edge_pass_1 = _make_edge_pass(64, 8)
_edge_pass_2 = _make_edge_pass(128, 128)


def kernel(x, edge_index, W1, att_src1, att_dst1, b1, W2, att_src2,
           att_dst2, b2):
    f32 = jnp.float32
    heads1, ch1 = att_src1.shape
    d1 = heads1 * ch1

    xp = jnp.zeros((NPAD, x.shape[1]), f32).at[:N].set(x)
    loops = jnp.arange(N, dtype=jnp.int32)
    pad_idx = jnp.full((P - E - N,), N, dtype=jnp.int32)
    src = jnp.concatenate([edge_index[0].astype(jnp.int32), loops, pad_idx])
    dst = jnp.concatenate([edge_index[1].astype(jnp.int32), loops, pad_idx])
    sd = jnp.stack([src.reshape(NW, KCH, C), dst.reshape(NW, KCH, C)],
                   axis=2)
    tail = jnp.full((NW, KAL - KCH, 2, C), N, dtype=jnp.int32)
    edges = jnp.concatenate([sd, tail], axis=1).reshape(NW * KAL, 2, C)

    eye_h = jnp.eye(heads1, dtype=f32)
    m_src1 = (att_src1[:, :, None] * eye_h[:, None, :]).reshape(d1, heads1)
    m_dst1 = (att_dst1[:, :, None] * eye_h[:, None, :]).reshape(d1, heads1)
    g1c = jnp.concatenate(
        [jnp.eye(d1, dtype=f32), m_src1, jnp.zeros((d1, 16 - heads1), f32)],
        axis=1)
    g1d = jnp.concatenate([m_dst1, jnp.zeros((d1, 16 - heads1), f32)], axis=1)

    d2 = W2.shape[1]
    g2c = jnp.concatenate(
        [jnp.eye(d2, dtype=f32), att_src2.T, jnp.zeros((d2, 15), f32)],
        axis=1)
    g2d = jnp.concatenate([att_dst2.T, jnp.zeros((d2, 15), f32)], axis=1)

    r1 = jnp.concatenate(
        [jnp.repeat(eye_h, ch1, axis=1), jnp.zeros((16 - heads1, d1), f32)],
        axis=0)
    r2 = jnp.zeros((16, d2), f32).at[0].set(1.0)

    z80 = jnp.zeros((NACC // NS, d1 + 16), f32)
    z144 = jnp.zeros((NACC // NS, d2 + 16), f32)

    th1, tad1 = _tc_prep(xp, W1, g1c, g1d)
    ua1, ub1 = _edge_pass_1(th1, tad1, edges, z80)

    th2, tad2 = _tc_mid(ua1, ub1, r1, b1.reshape(1, d1), W2, g2c, g2d)
    ua2, ub2 = _edge_pass_2(th2, tad2, edges, z144)

    return _tc_final(ua2[:N], ub2[:N], r2, b2.reshape(1, d2))
